# baseline (speedup 1.0000x reference)
import functools

import jax
import jax.numpy as jnp
from jax import lax
from jax.experimental import pallas as pl
from jax.experimental.pallas import tpu as pltpu
from jax.experimental.pallas import tpu_sc as plsc

f32 = jnp.float32
i32 = jnp.int32

N = 10000
E = 320000
F = 128
H = 64
G = 64
C = 10

NP = 10240
CHUNK = 128
CPT = 80
NCORES = 2
NSUB = 16
TILES = NCORES * NSUB
EP = TILES * CPT * CHUNK
ROWS_PT = NP // NSUB
RB = 1024


def _zero_zbuf(zbuf, width):
    def body(r, _):
        for j in range(width // 16):
            zbuf[r, pl.ds(j * 16, 16)] = jnp.zeros((16,), f32)
        return 0
    lax.fori_loop(0, 16, body, 0)


def _zero_acc_slice(zbuf, acc, s):
    base = s * ROWS_PT
    def body(j, _):
        pltpu.sync_copy(zbuf, acc.at[pl.ds(base + j * 16, 16)])
        return 0
    lax.fori_loop(0, ROWS_PT // 16, body, 0)


NB = 8
PRE = 4
ZROWS = 32


def _msg_body(g_hbm, src_hbm, dst_hbm, out_hbm, sidx, didx, rows, zbuf, acc,
              gsem, ssem):
    c = lax.axis_index("c")
    s = lax.axis_index("s")
    wid = c * NSUB + s
    base = s * ROWS_PT

    def zb(r, _):
        for j in range(H // 16):
            zbuf[r, pl.ds(j * 16, 16)] = jnp.zeros((16,), f32)
        return 0
    lax.fori_loop(0, ZROWS, zb, 0)

    def zacc(j, _):
        pltpu.sync_copy(zbuf, acc.at[pl.ds(base + j * ZROWS, ZROWS)])
        return 0
    lax.fori_loop(0, ROWS_PT // ZROWS, zacc, 0)
    pltpu.sync_copy(src_hbm.at[pl.ds(wid * CPT, CPT)], sidx)
    pltpu.sync_copy(dst_hbm.at[pl.ds(wid * CPT, CPT)], didx)
    plsc.subcore_barrier()

    def g_issue(ci, b):
        pltpu.async_copy(g_hbm.at[sidx.at[ci]], rows.at[b], gsem.at[b])

    def g_wait(ci, b):
        pltpu.make_async_copy(g_hbm.at[sidx.at[ci]], rows.at[b],
                              gsem.at[b]).wait()

    def s_issue(ci, b):
        pltpu.async_copy(rows.at[b], acc.at[didx.at[ci]], ssem.at[b],
                         add=True)

    def s_wait(ci, b):
        pltpu.make_async_copy(rows.at[b], acc.at[didx.at[ci]],
                              ssem.at[b]).wait()

    for b in range(PRE):
        g_issue(b, b)
    for cj in range(PRE):
        g_wait(cj, cj)
        s_issue(cj, cj)
        g_issue(cj + PRE, cj + PRE)

    def step(r, _):
        for k in range(NB):
            cj = r * NB + PRE + k
            b = (PRE + k) % NB
            rb = k % NB
            g_wait(cj, b)
            s_issue(cj, b)
            s_wait(cj - PRE, rb)
            g_issue(cj + PRE, rb)
        return 0
    lax.fori_loop(0, (CPT - 2 * PRE) // NB, step, 0)

    for k in range(PRE):
        cj = CPT - PRE + k
        b = (PRE + k) % NB
        g_wait(cj, b)
        s_issue(cj, b)
    for k in range(NB):
        s_wait(CPT - NB + k, k)

    plsc.subcore_barrier()
    pltpu.sync_copy(acc.at[pl.ds(base, ROWS_PT)],
                    out_hbm.at[c, pl.ds(base, ROWS_PT)])


def _deg_body(dst_hbm, out_hbm, didx, ones_v, zbuf, acc, sem):
    c = lax.axis_index("c")
    s = lax.axis_index("s")
    wid = c * NSUB + s
    def fill_ones(r, _):
        ones_v[r, :] = jnp.ones((16,), f32)
        return 0
    lax.fori_loop(0, CHUNK, fill_ones, 0)
    _zero_zbuf(zbuf, 16)
    _zero_acc_slice(zbuf, acc, s)
    pltpu.sync_copy(dst_hbm.at[pl.ds(wid * CPT, CPT)], didx)
    plsc.subcore_barrier()

    def step(ci, _):
        pltpu.sync_copy(ones_v, acc.at[didx.at[ci]], add=True)
        return 0
    lax.fori_loop(0, CPT, step, 0)

    plsc.subcore_barrier()
    base = s * ROWS_PT
    pltpu.sync_copy(acc.at[pl.ds(base, ROWS_PT)],
                    out_hbm.at[c, pl.ds(base, ROWS_PT)])


_SC_MESH = plsc.VectorSubcoreMesh(core_axis_name="c", subcore_axis_name="s")

_msg_call = functools.partial(
    pl.kernel,
    mesh=_SC_MESH,
    compiler_params=pltpu.CompilerParams(use_tc_tiling_on_sc=False),
    out_type=jax.ShapeDtypeStruct((NCORES, NP, H), f32),
    scratch_types=[
        pltpu.VMEM((CPT, CHUNK), i32),
        pltpu.VMEM((CPT, CHUNK), i32),
        pltpu.VMEM((NB, CHUNK, H), f32),
        pltpu.VMEM((ZROWS, H), f32),
        pltpu.VMEM_SHARED((NP, H), f32),
        pltpu.SemaphoreType.DMA((NB,)),
        pltpu.SemaphoreType.DMA((NB,)),
    ],
)(_msg_body)

_deg_call = functools.partial(
    pl.kernel,
    mesh=_SC_MESH,
    compiler_params=pltpu.CompilerParams(use_tc_tiling_on_sc=False),
    out_type=jax.ShapeDtypeStruct((NCORES, NP, 16), f32),
    scratch_types=[
        pltpu.VMEM((CPT, CHUNK), i32),
        pltpu.VMEM((CHUNK, 16), f32),
        pltpu.VMEM((16, 16), f32),
        pltpu.VMEM_SHARED((NP, 16), f32),
        pltpu.SemaphoreType.DMA,
    ],
)(_deg_body)


def _mm_body(x_ref, w_ref, o_ref):
    o_ref[...] = jnp.dot(x_ref[...], w_ref[...], preferred_element_type=f32)


_mm = pl.pallas_call(
    _mm_body,
    grid=(NP // RB,),
    in_specs=[pl.BlockSpec((RB, F), lambda i: (i, 0)),
              pl.BlockSpec((F, H), lambda i: (0, 0))],
    out_specs=pl.BlockSpec((RB, H), lambda i: (i, 0)),
    out_shape=jax.ShapeDtypeStruct((NP, H), f32),
)


def _dinv_g_body(degA_ref, degB_ref, h_ref, dinv_ref, g_ref):
    deg = degA_ref[:, :1] + degB_ref[:, :1] + 1.0
    dinv = lax.rsqrt(deg)
    dinv_ref[...] = dinv
    g_ref[...] = h_ref[...] * dinv


_dinv_g = pl.pallas_call(
    _dinv_g_body,
    grid=(NP // RB,),
    in_specs=[pl.BlockSpec((RB, 16), lambda i: (i, 0)),
              pl.BlockSpec((RB, 16), lambda i: (i, 0)),
              pl.BlockSpec((RB, H), lambda i: (i, 0))],
    out_specs=[pl.BlockSpec((RB, 1), lambda i: (i, 0)),
               pl.BlockSpec((RB, H), lambda i: (i, 0))],
    out_shape=[jax.ShapeDtypeStruct((NP, 1), f32),
               jax.ShapeDtypeStruct((NP, H), f32)],
)


def _layer_body(accA_ref, accB_ref, g_ref, dinv_ref, b_ref, w_ref, go_ref):
    dinv = dinv_ref[...]
    h = dinv * (accA_ref[...] + accB_ref[...] + g_ref[...]) + b_ref[...]
    h = jnp.maximum(h, 0.0)
    go_ref[...] = dinv * jnp.dot(h, w_ref[...], preferred_element_type=f32)


_layer = pl.pallas_call(
    _layer_body,
    grid=(NP // RB,),
    in_specs=[pl.BlockSpec((RB, H), lambda i: (i, 0)),
              pl.BlockSpec((RB, H), lambda i: (i, 0)),
              pl.BlockSpec((RB, H), lambda i: (i, 0)),
              pl.BlockSpec((RB, 1), lambda i: (i, 0)),
              pl.BlockSpec((1, H), lambda i: (0, 0)),
              pl.BlockSpec((H, H), lambda i: (0, 0))],
    out_specs=pl.BlockSpec((RB, H), lambda i: (i, 0)),
    out_shape=jax.ShapeDtypeStruct((NP, H), f32),
)


def _pool_body(accA_ref, accB_ref, g_ref, dinv_ref, b_ref, batch_ref,
               wc1_ref, bc1_ref, wc2_ref, bc2_ref, out_ref, seg_acc, cnt_acc):
    i = pl.program_id(0)
    dinv = dinv_ref[...]
    h = dinv * (accA_ref[...] + accB_ref[...] + g_ref[...]) + b_ref[...]
    h = jnp.maximum(h, 0.0)
    row = lax.broadcasted_iota(i32, (RB, 1), 0) + i * RB
    valid = row < N
    seg = lax.broadcasted_iota(i32, (1, G), 1)
    batch_i = batch_ref[...].astype(i32)
    onehot = jnp.where((batch_i == seg) & valid, 1.0, 0.0)
    contrib = lax.dot_general(onehot, h, (((0,), (0,)), ((), ())),
                              preferred_element_type=f32)
    csum = lax.dot_general(onehot, jnp.ones((RB, 1), f32),
                           (((0,), (0,)), ((), ())),
                           preferred_element_type=f32)

    @pl.when(i == 0)
    def _():
        seg_acc[...] = contrib
        cnt_acc[...] = csum
        out_ref[...] = jnp.zeros((G, C), f32)

    @pl.when(i > 0)
    def _():
        seg_acc[...] += contrib
        cnt_acc[...] += csum

    @pl.when(i == NP // RB - 1)
    def _():
        pooled = seg_acc[...] / jnp.maximum(cnt_acc[...], 1.0)
        z = jnp.maximum(
            jnp.dot(pooled, wc1_ref[...], preferred_element_type=f32)
            + bc1_ref[...], 0.0)
        out_ref[...] = (jnp.dot(z, wc2_ref[...], preferred_element_type=f32)
                        + bc2_ref[...])


_pool = pl.pallas_call(
    _pool_body,
    grid=(NP // RB,),
    in_specs=[pl.BlockSpec((RB, H), lambda i: (i, 0)),
              pl.BlockSpec((RB, H), lambda i: (i, 0)),
              pl.BlockSpec((RB, H), lambda i: (i, 0)),
              pl.BlockSpec((RB, 1), lambda i: (i, 0)),
              pl.BlockSpec((1, H), lambda i: (0, 0)),
              pl.BlockSpec((RB, 1), lambda i: (i, 0)),
              pl.BlockSpec((H, 32), lambda i: (0, 0)),
              pl.BlockSpec((1, 32), lambda i: (0, 0)),
              pl.BlockSpec((32, C), lambda i: (0, 0)),
              pl.BlockSpec((1, C), lambda i: (0, 0))],
    out_specs=pl.BlockSpec((G, C), lambda i: (0, 0)),
    out_shape=jax.ShapeDtypeStruct((G, C), f32),
    scratch_shapes=[pltpu.VMEM((G, H), f32), pltpu.VMEM((G, 1), f32)],
)


def kernel(x, edge_index, batch, W1, b1, W2, b2, W3, b3, Wc1, bc1, Wc2, bc2):
    src = edge_index[0].astype(i32)
    dst = edge_index[1].astype(i32)
    pad_k = jnp.arange(EP - E, dtype=i32)
    src_p = jnp.concatenate(
        [src, pad_k % 256]).reshape(TILES * CPT, CHUNK)
    dst_p = jnp.concatenate(
        [dst, N + pad_k % (NP - N)]).reshape(TILES * CPT, CHUNK)
    x_p = jnp.pad(x, ((0, NP - N), (0, 0)))
    batch_p = jnp.pad(batch.astype(f32), (0, NP - N)).reshape(NP, 1)

    deg2 = _deg_call(dst_p)
    h1pre = _mm(x_p, W1)
    dinv, g1 = _dinv_g(deg2[0], deg2[1], h1pre)
    a1 = _msg_call(g1, src_p, dst_p)
    g2 = _layer(a1[0], a1[1], g1, dinv, b1.reshape(1, H), W2)
    a2 = _msg_call(g2, src_p, dst_p)
    g3 = _layer(a2[0], a2[1], g2, dinv, b2.reshape(1, H), W3)
    a3 = _msg_call(g3, src_p, dst_p)
    out = _pool(a3[0], a3[1], g3, dinv, b3.reshape(1, H), batch_p,
                Wc1, bc1.reshape(1, 32), Wc2, bc2.reshape(1, C))
    return out


# speedup vs baseline: 1.3009x; 1.3009x over previous
import functools

import jax
import jax.numpy as jnp
from jax import lax
from jax.experimental import pallas as pl
from jax.experimental.pallas import tpu as pltpu
from jax.experimental.pallas import tpu_sc as plsc

f32 = jnp.float32
i32 = jnp.int32

N = 10000
E = 320000
F = 128
H = 64
G = 64
C = 10

NP = 10240
CHUNK = 128
CPT = 80
NCORES = 2
NSUB = 16
TILES = NCORES * NSUB
EP = TILES * CPT * CHUNK
ROWS_PT = NP // NSUB
RB = 1024


def _zero_zbuf(zbuf, width):
    def body(r, _):
        for j in range(width // 16):
            zbuf[r, pl.ds(j * 16, 16)] = jnp.zeros((16,), f32)
        return 0
    lax.fori_loop(0, 16, body, 0)


def _zero_acc_slice(zbuf, acc, s):
    base = s * ROWS_PT
    def body(j, _):
        pltpu.sync_copy(zbuf, acc.at[pl.ds(base + j * 16, 16)])
        return 0
    lax.fori_loop(0, ROWS_PT // 16, body, 0)


NBUF = 4


def _msg_body(g_hbm, src_hbm, dst_hbm, out_hbm, sidx, didx, rows, zbuf, acc,
              gsem):
    c = lax.axis_index("c")
    s = lax.axis_index("s")
    wid = c * NSUB + s
    base = s * ROWS_PT
    _zero_zbuf(zbuf, H)
    _zero_acc_slice(zbuf, acc, s)
    pltpu.sync_copy(src_hbm.at[pl.ds(wid * CPT, CPT)], sidx)
    pltpu.sync_copy(dst_hbm.at[pl.ds(wid * CPT, CPT)], didx)
    plsc.subcore_barrier()

    for b in range(NBUF):
        pltpu.async_copy(g_hbm.at[sidx.at[b]], rows.at[b], gsem.at[b])

    def step(ci4, _):
        for b in range(NBUF):
            ci = ci4 * NBUF + b
            pltpu.make_async_copy(g_hbm.at[sidx.at[ci]], rows.at[b],
                                  gsem.at[b]).wait()
            pltpu.sync_copy(rows.at[b], acc.at[didx.at[ci]], add=True)
            pltpu.async_copy(g_hbm.at[sidx.at[ci + NBUF]], rows.at[b],
                             gsem.at[b])
        return 0
    lax.fori_loop(0, CPT // NBUF - 1, step, 0)
    for b in range(NBUF):
        ci = CPT - NBUF + b
        pltpu.make_async_copy(g_hbm.at[sidx.at[ci]], rows.at[b],
                              gsem.at[b]).wait()
        pltpu.sync_copy(rows.at[b], acc.at[didx.at[ci]], add=True)

    plsc.subcore_barrier()
    pltpu.sync_copy(acc.at[pl.ds(base, ROWS_PT)],
                    out_hbm.at[c, pl.ds(base, ROWS_PT)])


def _deg_body(dst_hbm, out_hbm, didx, ones_v, zbuf, acc, sem):
    c = lax.axis_index("c")
    s = lax.axis_index("s")
    wid = c * NSUB + s
    def fill_ones(r, _):
        ones_v[r, :] = jnp.ones((16,), f32)
        return 0
    lax.fori_loop(0, CHUNK, fill_ones, 0)
    _zero_zbuf(zbuf, 16)
    _zero_acc_slice(zbuf, acc, s)
    pltpu.sync_copy(dst_hbm.at[pl.ds(wid * CPT, CPT)], didx)
    plsc.subcore_barrier()

    def step(ci, _):
        pltpu.sync_copy(ones_v, acc.at[didx.at[ci]], add=True)
        return 0
    lax.fori_loop(0, CPT, step, 0)

    plsc.subcore_barrier()
    base = s * ROWS_PT
    pltpu.sync_copy(acc.at[pl.ds(base, ROWS_PT)],
                    out_hbm.at[c, pl.ds(base, ROWS_PT)])


_SC_MESH = plsc.VectorSubcoreMesh(core_axis_name="c", subcore_axis_name="s")

_msg_call = functools.partial(
    pl.kernel,
    mesh=_SC_MESH,
    compiler_params=pltpu.CompilerParams(use_tc_tiling_on_sc=False),
    out_type=jax.ShapeDtypeStruct((NCORES, NP, H), f32),
    scratch_types=[
        pltpu.VMEM((CPT, CHUNK), i32),
        pltpu.VMEM((CPT, CHUNK), i32),
        pltpu.VMEM((NBUF, CHUNK, H), f32),
        pltpu.VMEM((16, H), f32),
        pltpu.VMEM_SHARED((NP, H), f32),
        pltpu.SemaphoreType.DMA((NBUF,)),
    ],
)(_msg_body)

_deg_call = functools.partial(
    pl.kernel,
    mesh=_SC_MESH,
    compiler_params=pltpu.CompilerParams(use_tc_tiling_on_sc=False),
    out_type=jax.ShapeDtypeStruct((NCORES, NP, 16), f32),
    scratch_types=[
        pltpu.VMEM((CPT, CHUNK), i32),
        pltpu.VMEM((CHUNK, 16), f32),
        pltpu.VMEM((16, 16), f32),
        pltpu.VMEM_SHARED((NP, 16), f32),
        pltpu.SemaphoreType.DMA,
    ],
)(_deg_body)


PN = NP // 2
PRB = RB // 2


def _head_body(xpk_ref, w_ref, dA_ref, dB_ref, dinvp_ref, g_ref):
    h = jnp.dot(xpk_ref[...], w_ref[...], preferred_element_type=f32)
    deg_e = dA_ref[:, 0:1] + dB_ref[:, 0:1] + 1.0
    deg_o = dA_ref[:, 16:17] + dB_ref[:, 16:17] + 1.0
    dinvp = jnp.concatenate(
        [jnp.broadcast_to(lax.rsqrt(deg_e), (PRB, H)),
         jnp.broadcast_to(lax.rsqrt(deg_o), (PRB, H))], axis=1)
    dinvp_ref[...] = dinvp
    g_ref[...] = h * dinvp


_head = pl.pallas_call(
    _head_body,
    grid=(PN // PRB,),
    in_specs=[pl.BlockSpec((PRB, 2 * F), lambda i: (i, 0)),
              pl.BlockSpec((2 * F, 2 * H), lambda i: (0, 0)),
              pl.BlockSpec((PRB, 32), lambda i: (i, 0)),
              pl.BlockSpec((PRB, 32), lambda i: (i, 0))],
    out_specs=[pl.BlockSpec((PRB, 2 * H), lambda i: (i, 0)),
               pl.BlockSpec((PRB, 2 * H), lambda i: (i, 0))],
    out_shape=[jax.ShapeDtypeStruct((PN, 2 * H), f32),
               jax.ShapeDtypeStruct((PN, 2 * H), f32)],
)


def _layer_body(accA_ref, accB_ref, g_ref, dinvp_ref, b_ref, w_ref, go_ref):
    dinvp = dinvp_ref[...]
    h = dinvp * (accA_ref[0] + accB_ref[0] + g_ref[...]) + b_ref[...]
    h = jnp.maximum(h, 0.0)
    go_ref[...] = dinvp * jnp.dot(h, w_ref[...], preferred_element_type=f32)


_layer = pl.pallas_call(
    _layer_body,
    grid=(PN // PRB,),
    in_specs=[pl.BlockSpec((1, PRB, 2 * H), lambda i: (0, i, 0)),
              pl.BlockSpec((1, PRB, 2 * H), lambda i: (1, i, 0)),
              pl.BlockSpec((PRB, 2 * H), lambda i: (i, 0)),
              pl.BlockSpec((PRB, 2 * H), lambda i: (i, 0)),
              pl.BlockSpec((1, 2 * H), lambda i: (0, 0)),
              pl.BlockSpec((2 * H, 2 * H), lambda i: (0, 0))],
    out_specs=pl.BlockSpec((PRB, 2 * H), lambda i: (i, 0)),
    out_shape=jax.ShapeDtypeStruct((PN, 2 * H), f32),
)


def _pool_body(accA_ref, accB_ref, g_ref, dinvp_ref, b_ref, be_ref, bo_ref,
               wc1_ref, bc1_ref, wc2_ref, bc2_ref, out_ref, seg_acc, cnt_acc):
    i = pl.program_id(0)
    dinvp = dinvp_ref[...]
    h = dinvp * (accA_ref[0] + accB_ref[0] + g_ref[...]) + b_ref[...]
    h = jnp.maximum(h, 0.0)
    prow = lax.broadcasted_iota(i32, (PRB, 1), 0) + i * PRB
    valid = prow < N // 2
    seg = lax.broadcasted_iota(i32, (1, G), 1)
    ohe = jnp.where((be_ref[...].astype(i32) == seg) & valid, 1.0, 0.0)
    oho = jnp.where((bo_ref[...].astype(i32) == seg) & valid, 1.0, 0.0)
    contrib = (
        lax.dot_general(ohe, h[:, :H], (((0,), (0,)), ((), ())),
                        preferred_element_type=f32)
        + lax.dot_general(oho, h[:, H:], (((0,), (0,)), ((), ())),
                          preferred_element_type=f32))
    csum = lax.dot_general(ohe + oho, jnp.ones((PRB, 1), f32),
                           (((0,), (0,)), ((), ())),
                           preferred_element_type=f32)

    @pl.when(i == 0)
    def _():
        seg_acc[...] = contrib
        cnt_acc[...] = csum
        out_ref[...] = jnp.zeros((G, C), f32)

    @pl.when(i > 0)
    def _():
        seg_acc[...] += contrib
        cnt_acc[...] += csum

    @pl.when(i == PN // PRB - 1)
    def _():
        pooled = seg_acc[...] / jnp.maximum(cnt_acc[...], 1.0)
        z = jnp.maximum(
            jnp.dot(pooled, wc1_ref[...], preferred_element_type=f32)
            + bc1_ref[...], 0.0)
        out_ref[...] = (jnp.dot(z, wc2_ref[...], preferred_element_type=f32)
                        + bc2_ref[...])


_pool = pl.pallas_call(
    _pool_body,
    grid=(PN // PRB,),
    in_specs=[pl.BlockSpec((1, PRB, 2 * H), lambda i: (0, i, 0)),
              pl.BlockSpec((1, PRB, 2 * H), lambda i: (1, i, 0)),
              pl.BlockSpec((PRB, 2 * H), lambda i: (i, 0)),
              pl.BlockSpec((PRB, 2 * H), lambda i: (i, 0)),
              pl.BlockSpec((1, 2 * H), lambda i: (0, 0)),
              pl.BlockSpec((PRB, 1), lambda i: (i, 0)),
              pl.BlockSpec((PRB, 1), lambda i: (i, 0)),
              pl.BlockSpec((H, 32), lambda i: (0, 0)),
              pl.BlockSpec((1, 32), lambda i: (0, 0)),
              pl.BlockSpec((32, C), lambda i: (0, 0)),
              pl.BlockSpec((1, C), lambda i: (0, 0))],
    out_specs=pl.BlockSpec((G, C), lambda i: (0, 0)),
    out_shape=jax.ShapeDtypeStruct((G, C), f32),
    scratch_shapes=[pltpu.VMEM((G, H), f32), pltpu.VMEM((G, 1), f32)],
)


def kernel(x, edge_index, batch, W1, b1, W2, b2, W3, b3, Wc1, bc1, Wc2, bc2):
    src = edge_index[0].astype(i32)
    dst = edge_index[1].astype(i32)
    pad_k = jnp.arange(EP - E, dtype=i32)
    src_p = jnp.concatenate(
        [src, pad_k % 256]).reshape(TILES * CPT, CHUNK)
    dst_p = jnp.concatenate(
        [dst, N + pad_k % (NP - N)]).reshape(TILES * CPT, CHUNK)
    xpk = jnp.pad(x, ((0, NP - N), (0, 0))).reshape(PN, 2 * F)
    eye2 = jnp.eye(2, dtype=f32)
    W1x = jnp.kron(eye2, W1)
    W2x = jnp.kron(eye2, W2)
    W3x = jnp.kron(eye2, W3)
    b1x = jnp.concatenate([b1, b1]).reshape(1, 2 * H)
    b2x = jnp.concatenate([b2, b2]).reshape(1, 2 * H)
    b3x = jnp.concatenate([b3, b3]).reshape(1, 2 * H)
    batch_p = jnp.pad(batch.astype(f32), (0, NP - N))
    be = batch_p[0::2].reshape(PN, 1)
    bo = batch_p[1::2].reshape(PN, 1)

    deg2 = _deg_call(dst_p)
    dApk = deg2[0].reshape(PN, 32)
    dBpk = deg2[1].reshape(PN, 32)
    dinvp, g1p = _head(xpk, W1x, dApk, dBpk)
    a1 = _msg_call(g1p.reshape(NP, H), src_p, dst_p).reshape(2, PN, 2 * H)
    g2p = _layer(a1, a1, g1p, dinvp, b1x, W2x)
    a2 = _msg_call(g2p.reshape(NP, H), src_p, dst_p).reshape(2, PN, 2 * H)
    g3p = _layer(a2, a2, g2p, dinvp, b2x, W3x)
    a3 = _msg_call(g3p.reshape(NP, H), src_p, dst_p).reshape(2, PN, 2 * H)
    out = _pool(a3, a3, g3p, dinvp, b3x, be, bo,
                Wc1, bc1.reshape(1, 32), Wc2, bc2.reshape(1, C))
    return out


# speedup vs baseline: 1.3507x; 1.0383x over previous
import functools

import jax
import jax.numpy as jnp
from jax import lax
from jax.experimental import pallas as pl
from jax.experimental.pallas import tpu as pltpu
from jax.experimental.pallas import tpu_sc as plsc

f32 = jnp.float32
i32 = jnp.int32

N = 10000
E = 320000
F = 128
H = 64
G = 64
C = 10

NP = 10240
CHUNK = 128
CPT = 80
NCORES = 2
NSUB = 16
TILES = NCORES * NSUB
EP = TILES * CPT * CHUNK
ROWS_PT = NP // NSUB
RB = 1024


def _zero_zbuf(zbuf, width):
    def body(r, _):
        for j in range(width // 16):
            zbuf[r, pl.ds(j * 16, 16)] = jnp.zeros((16,), f32)
        return 0
    lax.fori_loop(0, 16, body, 0)


def _zero_acc_slice(zbuf, acc, s):
    base = s * ROWS_PT
    def body(j, _):
        pltpu.sync_copy(zbuf, acc.at[pl.ds(base + j * 16, 16)])
        return 0
    lax.fori_loop(0, ROWS_PT // 16, body, 0)


NBUF = 4


def _msg_body(g_hbm, edges_hbm, out_hbm, sidx, didx, rows, zbuf, acc,
              gsem):
    c = lax.axis_index("c")
    s = lax.axis_index("s")
    wid = c * NSUB + s
    base = s * ROWS_PT
    _zero_zbuf(zbuf, H)
    _zero_acc_slice(zbuf, acc, s)
    pltpu.sync_copy(edges_hbm.at[0, pl.ds(wid * CPT, CPT)], sidx)
    pltpu.sync_copy(edges_hbm.at[1, pl.ds(wid * CPT, CPT)], didx)
    plsc.subcore_barrier()

    for b in range(NBUF):
        pltpu.async_copy(g_hbm.at[sidx.at[b]], rows.at[b], gsem.at[b])

    def step(ci4, _):
        for b in range(NBUF):
            ci = ci4 * NBUF + b
            pltpu.make_async_copy(g_hbm.at[sidx.at[ci]], rows.at[b],
                                  gsem.at[b]).wait()
            pltpu.sync_copy(rows.at[b], acc.at[didx.at[ci]], add=True)
            pltpu.async_copy(g_hbm.at[sidx.at[ci + NBUF]], rows.at[b],
                             gsem.at[b])
        return 0
    lax.fori_loop(0, CPT // NBUF - 1, step, 0)
    for b in range(NBUF):
        ci = CPT - NBUF + b
        pltpu.make_async_copy(g_hbm.at[sidx.at[ci]], rows.at[b],
                              gsem.at[b]).wait()
        pltpu.sync_copy(rows.at[b], acc.at[didx.at[ci]], add=True)

    plsc.subcore_barrier()
    pltpu.sync_copy(acc.at[pl.ds(base, ROWS_PT)],
                    out_hbm.at[c, pl.ds(base, ROWS_PT)])


def _deg_body(edges_hbm, out_hbm, didx, ones_v, zbuf, acc, sem):
    c = lax.axis_index("c")
    s = lax.axis_index("s")
    wid = c * NSUB + s
    def fill_ones(r, _):
        for j in range(H // 16):
            ones_v[r, pl.ds(j * 16, 16)] = jnp.ones((16,), f32)
        return 0
    lax.fori_loop(0, CHUNK, fill_ones, 0)
    _zero_zbuf(zbuf, H)
    _zero_acc_slice(zbuf, acc, s)
    pltpu.sync_copy(edges_hbm.at[1, pl.ds(wid * CPT, CPT)], didx)
    plsc.subcore_barrier()

    def step(ci, _):
        pltpu.sync_copy(ones_v, acc.at[didx.at[ci]], add=True)
        return 0
    lax.fori_loop(0, CPT, step, 0)

    plsc.subcore_barrier()
    base = s * ROWS_PT
    pltpu.sync_copy(acc.at[pl.ds(base, ROWS_PT)],
                    out_hbm.at[c, pl.ds(base, ROWS_PT)])


_SC_MESH = plsc.VectorSubcoreMesh(core_axis_name="c", subcore_axis_name="s")

_msg_call = functools.partial(
    pl.kernel,
    mesh=_SC_MESH,
    compiler_params=pltpu.CompilerParams(use_tc_tiling_on_sc=False),
    out_type=jax.ShapeDtypeStruct((NCORES, NP, H), f32),
    scratch_types=[
        pltpu.VMEM((CPT, CHUNK), i32),
        pltpu.VMEM((CPT, CHUNK), i32),
        pltpu.VMEM((NBUF, CHUNK, H), f32),
        pltpu.VMEM((16, H), f32),
        pltpu.VMEM_SHARED((NP, H), f32),
        pltpu.SemaphoreType.DMA((NBUF,)),
    ],
)(_msg_body)

_deg_call = functools.partial(
    pl.kernel,
    mesh=_SC_MESH,
    compiler_params=pltpu.CompilerParams(use_tc_tiling_on_sc=False),
    out_type=jax.ShapeDtypeStruct((NCORES, NP, H), f32),
    scratch_types=[
        pltpu.VMEM((CPT, CHUNK), i32),
        pltpu.VMEM((CHUNK, H), f32),
        pltpu.VMEM((16, H), f32),
        pltpu.VMEM_SHARED((NP, H), f32),
        pltpu.SemaphoreType.DMA,
    ],
)(_deg_body)


PN = NP // 2
PRB = RB // 2


def _head_body(xpk_ref, w_ref, dA_ref, dB_ref, dinvp_ref, g_ref):
    h = jnp.dot(xpk_ref[...], w_ref[...], preferred_element_type=f32)
    dinvp = lax.rsqrt(dA_ref[0] + dB_ref[0] + 1.0)
    dinvp_ref[...] = dinvp
    g_ref[...] = h * dinvp


_head = pl.pallas_call(
    _head_body,
    grid=(PN // PRB,),
    in_specs=[pl.BlockSpec((PRB, 2 * F), lambda i: (i, 0)),
              pl.BlockSpec((2 * F, 2 * H), lambda i: (0, 0)),
              pl.BlockSpec((1, PRB, 2 * H), lambda i: (0, i, 0)),
              pl.BlockSpec((1, PRB, 2 * H), lambda i: (1, i, 0))],
    out_specs=[pl.BlockSpec((PRB, 2 * H), lambda i: (i, 0)),
               pl.BlockSpec((PRB, 2 * H), lambda i: (i, 0))],
    out_shape=[jax.ShapeDtypeStruct((PN, 2 * H), f32),
               jax.ShapeDtypeStruct((PN, 2 * H), f32)],
)


def _layer_body(accA_ref, accB_ref, g_ref, dinvp_ref, b_ref, w_ref, go_ref):
    dinvp = dinvp_ref[...]
    h = dinvp * (accA_ref[0] + accB_ref[0] + g_ref[...]) + b_ref[...]
    h = jnp.maximum(h, 0.0)
    go_ref[...] = dinvp * jnp.dot(h, w_ref[...], preferred_element_type=f32)


_layer = pl.pallas_call(
    _layer_body,
    grid=(PN // PRB,),
    in_specs=[pl.BlockSpec((1, PRB, 2 * H), lambda i: (0, i, 0)),
              pl.BlockSpec((1, PRB, 2 * H), lambda i: (1, i, 0)),
              pl.BlockSpec((PRB, 2 * H), lambda i: (i, 0)),
              pl.BlockSpec((PRB, 2 * H), lambda i: (i, 0)),
              pl.BlockSpec((1, 2 * H), lambda i: (0, 0)),
              pl.BlockSpec((2 * H, 2 * H), lambda i: (0, 0))],
    out_specs=pl.BlockSpec((PRB, 2 * H), lambda i: (i, 0)),
    out_shape=jax.ShapeDtypeStruct((PN, 2 * H), f32),
)


def _pool_body(accA_ref, accB_ref, g_ref, dinvp_ref, b_ref, be_ref, bo_ref,
               wc1_ref, bc1_ref, wc2_ref, bc2_ref, out_ref, seg_acc, cnt_acc):
    i = pl.program_id(0)
    dinvp = dinvp_ref[...]
    h = dinvp * (accA_ref[0] + accB_ref[0] + g_ref[...]) + b_ref[...]
    h = jnp.maximum(h, 0.0)
    prow = lax.broadcasted_iota(i32, (PRB, 1), 0) + i * PRB
    valid = prow < N // 2
    seg = lax.broadcasted_iota(i32, (1, G), 1)
    ohe = jnp.where((be_ref[...].astype(i32) == seg) & valid, 1.0, 0.0)
    oho = jnp.where((bo_ref[...].astype(i32) == seg) & valid, 1.0, 0.0)
    contrib = (
        lax.dot_general(ohe, h[:, :H], (((0,), (0,)), ((), ())),
                        preferred_element_type=f32)
        + lax.dot_general(oho, h[:, H:], (((0,), (0,)), ((), ())),
                          preferred_element_type=f32))
    csum = lax.dot_general(ohe + oho, jnp.ones((PRB, 1), f32),
                           (((0,), (0,)), ((), ())),
                           preferred_element_type=f32)

    @pl.when(i == 0)
    def _():
        seg_acc[...] = contrib
        cnt_acc[...] = csum
        out_ref[...] = jnp.zeros((G, C), f32)

    @pl.when(i > 0)
    def _():
        seg_acc[...] += contrib
        cnt_acc[...] += csum

    @pl.when(i == PN // PRB - 1)
    def _():
        pooled = seg_acc[...] / jnp.maximum(cnt_acc[...], 1.0)
        z = jnp.maximum(
            jnp.dot(pooled, wc1_ref[...], preferred_element_type=f32)
            + bc1_ref[...], 0.0)
        out_ref[...] = (jnp.dot(z, wc2_ref[...], preferred_element_type=f32)
                        + bc2_ref[...])


_pool = pl.pallas_call(
    _pool_body,
    grid=(PN // PRB,),
    in_specs=[pl.BlockSpec((1, PRB, 2 * H), lambda i: (0, i, 0)),
              pl.BlockSpec((1, PRB, 2 * H), lambda i: (1, i, 0)),
              pl.BlockSpec((PRB, 2 * H), lambda i: (i, 0)),
              pl.BlockSpec((PRB, 2 * H), lambda i: (i, 0)),
              pl.BlockSpec((1, 2 * H), lambda i: (0, 0)),
              pl.BlockSpec((PRB, 1), lambda i: (i, 0)),
              pl.BlockSpec((PRB, 1), lambda i: (i, 0)),
              pl.BlockSpec((H, 32), lambda i: (0, 0)),
              pl.BlockSpec((1, 32), lambda i: (0, 0)),
              pl.BlockSpec((32, C), lambda i: (0, 0)),
              pl.BlockSpec((1, C), lambda i: (0, 0))],
    out_specs=pl.BlockSpec((G, C), lambda i: (0, 0)),
    out_shape=jax.ShapeDtypeStruct((G, C), f32),
    scratch_shapes=[pltpu.VMEM((G, H), f32), pltpu.VMEM((G, 1), f32)],
)


def kernel(x, edge_index, batch, W1, b1, W2, b2, W3, b3, Wc1, bc1, Wc2, bc2):
    pad_k = jnp.arange(EP - E, dtype=i32)
    pads = jnp.stack([pad_k % 256,
                      N + pad_k % (NP - N)]).reshape(2, (EP - E) // CHUNK,
                                                     CHUNK)
    edges_p = jnp.concatenate(
        [edge_index.astype(i32).reshape(2, E // CHUNK, CHUNK), pads], axis=1)
    xpk = jnp.pad(x, ((0, NP - N), (0, 0))).reshape(PN, 2 * F)
    eye2 = jnp.eye(2, dtype=f32)
    W1x = jnp.kron(eye2, W1)
    W2x = jnp.kron(eye2, W2)
    W3x = jnp.kron(eye2, W3)
    b1x = jnp.concatenate([b1, b1]).reshape(1, 2 * H)
    b2x = jnp.concatenate([b2, b2]).reshape(1, 2 * H)
    b3x = jnp.concatenate([b3, b3]).reshape(1, 2 * H)
    batch_p = jnp.pad(batch.astype(f32), (0, NP - N))
    be = batch_p[0::2].reshape(PN, 1)
    bo = batch_p[1::2].reshape(PN, 1)

    degp = _deg_call(edges_p).reshape(2, PN, 2 * H)
    dinvp, g1p = _head(xpk, W1x, degp, degp)
    a1 = _msg_call(g1p.reshape(NP, H), edges_p).reshape(2, PN, 2 * H)
    g2p = _layer(a1, a1, g1p, dinvp, b1x, W2x)
    a2 = _msg_call(g2p.reshape(NP, H), edges_p).reshape(2, PN, 2 * H)
    g3p = _layer(a2, a2, g2p, dinvp, b2x, W3x)
    a3 = _msg_call(g3p.reshape(NP, H), edges_p).reshape(2, PN, 2 * H)
    out = _pool(a3, a3, g3p, dinvp, b3x, be, bo,
                Wc1, bc1.reshape(1, 32), Wc2, bc2.reshape(1, C))
    return out


# speedup vs baseline: 1.4591x; 1.0803x over previous
import functools

import jax
import jax.numpy as jnp
from jax import lax
from jax.experimental import pallas as pl
from jax.experimental.pallas import tpu as pltpu
from jax.experimental.pallas import tpu_sc as plsc

f32 = jnp.float32
i32 = jnp.int32

N = 10000
E = 320000
F = 128
H = 64
G = 64
C = 10

NP = 10240
CHUNK = 128
CPT = 80
NCORES = 2
NSUB = 16
TILES = NCORES * NSUB
EP = TILES * CPT * CHUNK
ROWS_PT = NP // NSUB
RB = 1024


def _zero_zbuf(zbuf, width):
    def body(r, _):
        for j in range(width // 16):
            zbuf[r, pl.ds(j * 16, 16)] = jnp.zeros((16,), f32)
        return 0
    lax.fori_loop(0, 16, body, 0)


def _zero_acc_slice(zbuf, acc, s):
    base = s * ROWS_PT
    def body(j, _):
        pltpu.sync_copy(zbuf, acc.at[pl.ds(base + j * 16, 16)])
        return 0
    lax.fori_loop(0, ROWS_PT // 16, body, 0)


NBUF = 4


def _msg_body(g_hbm, edges_hbm, out_hbm, sidx, didx, rows, zbuf, acc,
              gsem):
    c = lax.axis_index("c")
    s = lax.axis_index("s")
    wid = c * NSUB + s
    base = s * ROWS_PT
    _zero_zbuf(zbuf, H)
    _zero_acc_slice(zbuf, acc, s)
    pltpu.sync_copy(edges_hbm.at[0, pl.ds(wid * CPT, CPT)], sidx)
    pltpu.sync_copy(edges_hbm.at[1, pl.ds(wid * CPT, CPT)], didx)
    plsc.subcore_barrier()

    for b in range(NBUF):
        pltpu.async_copy(g_hbm.at[sidx.at[b]], rows.at[b], gsem.at[b])

    def step(ci4, _):
        for b in range(NBUF):
            ci = ci4 * NBUF + b
            pltpu.make_async_copy(g_hbm.at[sidx.at[ci]], rows.at[b],
                                  gsem.at[b]).wait()
            pltpu.sync_copy(rows.at[b], acc.at[didx.at[ci]], add=True)
            pltpu.async_copy(g_hbm.at[sidx.at[ci + NBUF]], rows.at[b],
                             gsem.at[b])
        return 0
    lax.fori_loop(0, CPT // NBUF - 1, step, 0)
    for b in range(NBUF):
        ci = CPT - NBUF + b
        pltpu.make_async_copy(g_hbm.at[sidx.at[ci]], rows.at[b],
                              gsem.at[b]).wait()
        pltpu.sync_copy(rows.at[b], acc.at[didx.at[ci]], add=True)

    plsc.subcore_barrier()
    pltpu.sync_copy(acc.at[pl.ds(base, ROWS_PT)],
                    out_hbm.at[c, pl.ds(base, ROWS_PT)])


def _deg_body(edges_hbm, out_hbm, didx, ones_v, zbuf, acc, sem):
    c = lax.axis_index("c")
    s = lax.axis_index("s")
    wid = c * NSUB + s
    def fill_ones(r, _):
        ones_v[r, :] = jnp.ones((16,), f32)
        return 0
    lax.fori_loop(0, CHUNK, fill_ones, 0)
    _zero_zbuf(zbuf, 16)
    _zero_acc_slice(zbuf, acc, s)
    pltpu.sync_copy(edges_hbm.at[1, pl.ds(wid * CPT, CPT)], didx)
    plsc.subcore_barrier()

    def step(ci, _):
        pltpu.sync_copy(ones_v, acc.at[didx.at[ci]], add=True)
        return 0
    lax.fori_loop(0, CPT, step, 0)

    plsc.subcore_barrier()
    base = s * ROWS_PT
    pltpu.sync_copy(acc.at[pl.ds(base, ROWS_PT)],
                    out_hbm.at[c, pl.ds(base, ROWS_PT)])


_SC_MESH = plsc.VectorSubcoreMesh(core_axis_name="c", subcore_axis_name="s")

_msg_call = functools.partial(
    pl.kernel,
    mesh=_SC_MESH,
    compiler_params=pltpu.CompilerParams(use_tc_tiling_on_sc=False),
    out_type=jax.ShapeDtypeStruct((NCORES, NP, H), f32),
    scratch_types=[
        pltpu.VMEM((CPT, CHUNK), i32),
        pltpu.VMEM((CPT, CHUNK), i32),
        pltpu.VMEM((NBUF, CHUNK, H), f32),
        pltpu.VMEM((16, H), f32),
        pltpu.VMEM_SHARED((NP, H), f32),
        pltpu.SemaphoreType.DMA((NBUF,)),
    ],
)(_msg_body)

_deg_call = functools.partial(
    pl.kernel,
    mesh=_SC_MESH,
    compiler_params=pltpu.CompilerParams(use_tc_tiling_on_sc=False),
    out_type=jax.ShapeDtypeStruct((NCORES, NP, 16), f32),
    scratch_types=[
        pltpu.VMEM((CPT, CHUNK), i32),
        pltpu.VMEM((CHUNK, 16), f32),
        pltpu.VMEM((16, 16), f32),
        pltpu.VMEM_SHARED((NP, 16), f32),
        pltpu.SemaphoreType.DMA,
    ],
)(_deg_body)


PN = NP // 2
PRB = RB // 2


DRB = PRB // 4


def _head_body(xpk_ref, w_ref, dA_ref, dB_ref, dinvp_ref, g_ref):
    h = jnp.dot(xpk_ref[...], w_ref[...], preferred_element_type=f32)
    d8 = dA_ref[0] + dB_ref[0]
    riota = lax.broadcasted_iota(i32, (PRB, 2 * H), 0)
    ciota = lax.broadcasted_iota(i32, (PRB, 2 * H), 1)
    expand = jnp.where(ciota[:, :DRB] == riota[:, :DRB] // 4, 1.0, 0.0)
    y = jnp.dot(expand, d8, preferred_element_type=f32)
    sel = 2 * (riota % 4) + jnp.where(ciota >= H, 1, 0)
    deg = jnp.zeros((PRB, 2 * H), f32)
    for t in range(8):
        deg = deg + jnp.where(sel == t,
                              jnp.broadcast_to(y[:, 16 * t:16 * t + 1],
                                               (PRB, 2 * H)), 0.0)
    dinvp = lax.rsqrt(deg + 1.0)
    dinvp_ref[...] = dinvp
    g_ref[...] = h * dinvp


_head = pl.pallas_call(
    _head_body,
    grid=(PN // PRB,),
    in_specs=[pl.BlockSpec((PRB, 2 * F), lambda i: (i, 0)),
              pl.BlockSpec((2 * F, 2 * H), lambda i: (0, 0)),
              pl.BlockSpec((1, DRB, 2 * H), lambda i: (0, i, 0)),
              pl.BlockSpec((1, DRB, 2 * H), lambda i: (1, i, 0))],
    out_specs=[pl.BlockSpec((PRB, 2 * H), lambda i: (i, 0)),
               pl.BlockSpec((PRB, 2 * H), lambda i: (i, 0))],
    out_shape=[jax.ShapeDtypeStruct((PN, 2 * H), f32),
               jax.ShapeDtypeStruct((PN, 2 * H), f32)],
)


def _layer_body(accA_ref, accB_ref, g_ref, dinvp_ref, b_ref, w_ref, go_ref):
    dinvp = dinvp_ref[...]
    h = dinvp * (accA_ref[0] + accB_ref[0] + g_ref[...]) + b_ref[...]
    h = jnp.maximum(h, 0.0)
    go_ref[...] = dinvp * jnp.dot(h, w_ref[...], preferred_element_type=f32)


_layer = pl.pallas_call(
    _layer_body,
    grid=(PN // PRB,),
    in_specs=[pl.BlockSpec((1, PRB, 2 * H), lambda i: (0, i, 0)),
              pl.BlockSpec((1, PRB, 2 * H), lambda i: (1, i, 0)),
              pl.BlockSpec((PRB, 2 * H), lambda i: (i, 0)),
              pl.BlockSpec((PRB, 2 * H), lambda i: (i, 0)),
              pl.BlockSpec((1, 2 * H), lambda i: (0, 0)),
              pl.BlockSpec((2 * H, 2 * H), lambda i: (0, 0))],
    out_specs=pl.BlockSpec((PRB, 2 * H), lambda i: (i, 0)),
    out_shape=jax.ShapeDtypeStruct((PN, 2 * H), f32),
)


def _pool_body(accA_ref, accB_ref, g_ref, dinvp_ref, b_ref, be_ref, bo_ref,
               wc1_ref, bc1_ref, wc2_ref, bc2_ref, out_ref, seg_acc, cnt_acc):
    i = pl.program_id(0)
    dinvp = dinvp_ref[...]
    h = dinvp * (accA_ref[0] + accB_ref[0] + g_ref[...]) + b_ref[...]
    h = jnp.maximum(h, 0.0)
    prow = lax.broadcasted_iota(i32, (PRB, 1), 0) + i * PRB
    valid = prow < N // 2
    seg = lax.broadcasted_iota(i32, (1, G), 1)
    ohe = jnp.where((be_ref[...].astype(i32) == seg) & valid, 1.0, 0.0)
    oho = jnp.where((bo_ref[...].astype(i32) == seg) & valid, 1.0, 0.0)
    contrib = (
        lax.dot_general(ohe, h[:, :H], (((0,), (0,)), ((), ())),
                        preferred_element_type=f32)
        + lax.dot_general(oho, h[:, H:], (((0,), (0,)), ((), ())),
                          preferred_element_type=f32))
    csum = lax.dot_general(ohe + oho, jnp.ones((PRB, 1), f32),
                           (((0,), (0,)), ((), ())),
                           preferred_element_type=f32)

    @pl.when(i == 0)
    def _():
        seg_acc[...] = contrib
        cnt_acc[...] = csum
        out_ref[...] = jnp.zeros((G, C), f32)

    @pl.when(i > 0)
    def _():
        seg_acc[...] += contrib
        cnt_acc[...] += csum

    @pl.when(i == PN // PRB - 1)
    def _():
        pooled = seg_acc[...] / jnp.maximum(cnt_acc[...], 1.0)
        z = jnp.maximum(
            jnp.dot(pooled, wc1_ref[...], preferred_element_type=f32)
            + bc1_ref[...], 0.0)
        out_ref[...] = (jnp.dot(z, wc2_ref[...], preferred_element_type=f32)
                        + bc2_ref[...])


_pool = pl.pallas_call(
    _pool_body,
    grid=(PN // PRB,),
    in_specs=[pl.BlockSpec((1, PRB, 2 * H), lambda i: (0, i, 0)),
              pl.BlockSpec((1, PRB, 2 * H), lambda i: (1, i, 0)),
              pl.BlockSpec((PRB, 2 * H), lambda i: (i, 0)),
              pl.BlockSpec((PRB, 2 * H), lambda i: (i, 0)),
              pl.BlockSpec((1, 2 * H), lambda i: (0, 0)),
              pl.BlockSpec((PRB, 1), lambda i: (i, 0)),
              pl.BlockSpec((PRB, 1), lambda i: (i, 0)),
              pl.BlockSpec((H, 32), lambda i: (0, 0)),
              pl.BlockSpec((1, 32), lambda i: (0, 0)),
              pl.BlockSpec((32, C), lambda i: (0, 0)),
              pl.BlockSpec((1, C), lambda i: (0, 0))],
    out_specs=pl.BlockSpec((G, C), lambda i: (0, 0)),
    out_shape=jax.ShapeDtypeStruct((G, C), f32),
    scratch_shapes=[pltpu.VMEM((G, H), f32), pltpu.VMEM((G, 1), f32)],
)


def kernel(x, edge_index, batch, W1, b1, W2, b2, W3, b3, Wc1, bc1, Wc2, bc2):
    pad_k = jnp.arange(EP - E, dtype=i32)
    pads = jnp.stack([pad_k % 256,
                      N + pad_k % (NP - N)]).reshape(2, (EP - E) // CHUNK,
                                                     CHUNK)
    edges_p = jnp.concatenate(
        [edge_index.astype(i32).reshape(2, E // CHUNK, CHUNK), pads], axis=1)
    xpk = jnp.pad(x, ((0, NP - N), (0, 0))).reshape(PN, 2 * F)
    eye2 = jnp.eye(2, dtype=f32)
    W1x = jnp.kron(eye2, W1)
    W2x = jnp.kron(eye2, W2)
    W3x = jnp.kron(eye2, W3)
    b1x = jnp.concatenate([b1, b1]).reshape(1, 2 * H)
    b2x = jnp.concatenate([b2, b2]).reshape(1, 2 * H)
    b3x = jnp.concatenate([b3, b3]).reshape(1, 2 * H)
    batch_p = jnp.pad(batch.astype(f32), (0, NP - N))
    be = batch_p[0::2].reshape(PN, 1)
    bo = batch_p[1::2].reshape(PN, 1)

    degp = _deg_call(edges_p).reshape(2, NP // 8, 2 * H)
    dinvp, g1p = _head(xpk, W1x, degp, degp)
    a1 = _msg_call(g1p.reshape(NP, H), edges_p).reshape(2, PN, 2 * H)
    g2p = _layer(a1, a1, g1p, dinvp, b1x, W2x)
    a2 = _msg_call(g2p.reshape(NP, H), edges_p).reshape(2, PN, 2 * H)
    g3p = _layer(a2, a2, g2p, dinvp, b2x, W3x)
    a3 = _msg_call(g3p.reshape(NP, H), edges_p).reshape(2, PN, 2 * H)
    out = _pool(a3, a3, g3p, dinvp, b3x, be, bo,
                Wc1, bc1.reshape(1, 32), Wc2, bc2.reshape(1, C))
    return out
